# Initial kernel scaffold; baseline (speedup 1.0000x reference)
import jax
import jax.numpy as jnp
from jax.experimental import pallas as pl


def kernel(x_cat, price, tables, W_price, b_price, W1, b1, W2, b2, Wt1, bt1, Wt2, bt2):
    raise NotImplementedError("write your pallas kernel here")


# speedup vs baseline: 6.4929x; 6.4929x over previous
import functools

import jax
import jax.numpy as jnp
from jax import lax
from jax.experimental import pallas as pl
from jax.experimental.pallas import tpu as pltpu
from jax.experimental.pallas import tpu_sc as plsc


def _build_sc_gather(F, B, V, D, NW):
    b_per_w = B // NW
    CH = 128
    n_ch = b_per_w // CH
    mesh = plsc.VectorSubcoreMesh(core_axis_name="c", subcore_axis_name="s")

    @functools.partial(
        pl.kernel,
        mesh=mesh,
        out_type=jax.ShapeDtypeStruct((F, B, D), jnp.float32),
        compiler_params=pltpu.CompilerParams(use_tc_tiling_on_sc=False),
        scratch_types=[
            pltpu.VMEM((CH,), jnp.int32),
            pltpu.VMEM((CH, D), jnp.float32),
            pltpu.SemaphoreType.DMA,
        ],
    )
    def sc_gather(xcat_hbm, tflat_hbm, out_hbm, idx_v, rows_v, sem):
        wid = lax.axis_index("s") * 2 + lax.axis_index("c")
        base = wid * b_per_w

        def f_body(f, carry):
            def c_body(c, carry2):
                off = base + c * CH
                pltpu.sync_copy(xcat_hbm.at[f, pl.ds(off, CH)], idx_v)
                fV = f * V
                for g in range(CH // 16):
                    idx_v[pl.ds(g * 16, 16)] = idx_v[pl.ds(g * 16, 16)] + fV
                pltpu.async_copy(tflat_hbm.at[idx_v], rows_v, sem).wait()
                pltpu.sync_copy(rows_v, out_hbm.at[f, pl.ds(off, CH), :])
                return carry2
            return lax.fori_loop(0, n_ch, c_body, carry)

        lax.fori_loop(0, F, f_body, 0)

    return sc_gather


def _build_tc_dense(F, B, D, BBLK):
    NF = F + 1
    NP = NF * (NF - 1) // 2
    CS_ROWS = NP + 32 + 1

    def tc_dense(emb_ref, price_ref, wp_ref, bp_ref, w1_ref, b1_ref,
                 w2_ref, b2_ref, wt1_ref, bt1_ref, wt2_ref, bt2_ref,
                 out_ref, cs_ref):
        price = price_ref[:, :]
        de = wp_ref[:, :] * price + bp_ref[:, :]
        h = jnp.maximum(
            lax.dot_general(w1_ref[:, :], de, (((1,), (0,)), ((), ())),
                            preferred_element_type=jnp.float32)
            + b1_ref[:, :], 0.0)
        di = jnp.maximum(
            lax.dot_general(w2_ref[:, :], h, (((1,), (0,)), ((), ())),
                            preferred_element_type=jnp.float32)
            + b2_ref[:, :], 0.0)
        feats = [jnp.swapaxes(emb_ref[i], 0, 1) for i in range(F)] + [de]
        p = 0
        for i in range(NF):
            fi = feats[i]
            for j in range(i + 1, NF):
                s = jnp.sum(fi * feats[j], axis=0, keepdims=True)
                cs_ref[pl.ds(p, 1), :] = s
                p += 1
        cs_ref[pl.ds(NP, 32), :] = di
        cs_ref[pl.ds(NP + 32, 1), :] = jnp.zeros((1, BBLK), jnp.float32)
        t = jnp.maximum(
            lax.dot_general(wt1_ref[:, :], cs_ref[:, :],
                            (((1,), (0,)), ((), ())),
                            preferred_element_type=jnp.float32)
            + bt1_ref[:, :], 0.0)
        o = lax.dot_general(wt2_ref[:, :], t, (((1,), (0,)), ((), ())),
                            preferred_element_type=jnp.float32) \
            + bt2_ref[:, :]
        out_ref[:, :] = o

    grid = (B // BBLK,)
    full = lambda shp: pl.BlockSpec(shp, lambda i: (0, 0))
    return pl.pallas_call(
        tc_dense,
        grid=grid,
        in_specs=[
            pl.BlockSpec((F, BBLK, D), lambda i: (0, i, 0)),
            pl.BlockSpec((1, BBLK), lambda i: (0, i)),
            full((D, 1)), full((D, 1)),
            full((64, D)), full((64, 1)),
            full((32, 64)), full((32, 1)),
            full((32, CS_ROWS)), full((32, 1)),
            full((1, 32)), full((1, 1)),
        ],
        out_specs=pl.BlockSpec((1, BBLK), lambda i: (0, i)),
        out_shape=jax.ShapeDtypeStruct((1, B), jnp.float32),
        scratch_shapes=[pltpu.VMEM((CS_ROWS, BBLK), jnp.float32)],
    )


def kernel(x_cat, price, tables, W_price, b_price, W1, b1, W2, b2,
           Wt1, bt1, Wt2, bt2):
    F, B = x_cat.shape
    _, V, D = tables.shape
    NW = 32
    BBLK = 512

    tflat = tables.reshape(F * V, D)
    sc_gather = _build_sc_gather(F, B, V, D, NW)
    emb_t = sc_gather(x_cat.astype(jnp.int32), tflat)

    NP = (F + 1) * F // 2
    Wt1p = jnp.pad(Wt1, ((0, 0), (0, 1)))
    tc = _build_tc_dense(F, B, D, BBLK)
    out2d = tc(emb_t, price.reshape(1, B), W_price, b_price.reshape(D, 1),
               W1, b1.reshape(64, 1), W2, b2.reshape(32, 1),
               Wt1p, bt1.reshape(32, 1), Wt2, bt2.reshape(1, 1))
    return out2d.reshape(B)


# speedup vs baseline: 7.0719x; 1.0892x over previous
import functools

import jax
import jax.numpy as jnp
from jax import lax
from jax.experimental import pallas as pl
from jax.experimental.pallas import tpu as pltpu
from jax.experimental.pallas import tpu_sc as plsc


def _build_sc_gather(F, B, V, D, NW):
    b_per_w = B // NW
    CH = 128
    n_ch = b_per_w // CH
    mesh = plsc.VectorSubcoreMesh(core_axis_name="c", subcore_axis_name="s")

    @functools.partial(
        pl.kernel,
        mesh=mesh,
        out_type=jax.ShapeDtypeStruct((F, B, D), jnp.float32),
        compiler_params=pltpu.CompilerParams(use_tc_tiling_on_sc=False),
        scratch_types=[
            pltpu.VMEM((2, n_ch, CH), jnp.int32),
            pltpu.VMEM((2, b_per_w, D), jnp.float32),
            pltpu.SemaphoreType.DMA((2,)),
            pltpu.SemaphoreType.DMA((2,)),
            pltpu.SemaphoreType.DMA((2,)),
        ],
    )
    def sc_gather(gidx_hbm, tflat_hbm, out_hbm, idx_v, rows_v,
                  sem_i, sem_g, sem_o):
        wid = lax.axis_index("s") * 2 + lax.axis_index("c")
        base = wid * b_per_w
        cbase = wid * n_ch

        def idx_copy(f):
            s = f % 2
            return pltpu.make_async_copy(
                gidx_hbm.at[f, pl.ds(cbase, n_ch), :], idx_v.at[s],
                sem_i.at[s])

        def gathers(f):
            s = f % 2
            return [pltpu.make_async_copy(
                tflat_hbm.at[idx_v.at[s, c]],
                rows_v.at[s, pl.ds(c * CH, CH), :], sem_g.at[s])
                for c in range(n_ch)]

        def out_copy(f):
            s = f % 2
            return pltpu.make_async_copy(
                rows_v.at[s], out_hbm.at[f, pl.ds(base, b_per_w), :],
                sem_o.at[s])

        idx_copy(0).start()
        for f in range(F):
            idx_copy(f).wait()
            if f + 1 < F:
                idx_copy(f + 1).start()
            if f >= 2:
                out_copy(f - 2).wait()
            gs = gathers(f)
            for g in gs:
                g.start()
            if f >= 1:
                for g in gathers(f - 1):
                    g.wait()
                out_copy(f - 1).start()
        for g in gathers(F - 1):
            g.wait()
        out_copy(F - 1).start()
        out_copy(F - 2).wait()
        out_copy(F - 1).wait()

    return sc_gather


def _build_tc_dense(F, B, D, BBLK):
    NF = F + 1
    NP = NF * (NF - 1) // 2
    CS_ROWS = NP + 32 + 1

    def tc_dense(emb_ref, price_ref, wp_ref, bp_ref, w1_ref, b1_ref,
                 w2_ref, b2_ref, wt1_ref, bt1_ref, wt2_ref, bt2_ref,
                 out_ref, cs_ref):
        price = price_ref[:, :]
        de = wp_ref[:, :] * price + bp_ref[:, :]
        h = jnp.maximum(
            lax.dot_general(w1_ref[:, :], de, (((1,), (0,)), ((), ())),
                            preferred_element_type=jnp.float32)
            + b1_ref[:, :], 0.0)
        di = jnp.maximum(
            lax.dot_general(w2_ref[:, :], h, (((1,), (0,)), ((), ())),
                            preferred_element_type=jnp.float32)
            + b2_ref[:, :], 0.0)
        feats = [jnp.swapaxes(emb_ref[i], 0, 1) for i in range(F)] + [de]
        p = 0
        for i in range(NF):
            fi = feats[i]
            for j in range(i + 1, NF):
                s = jnp.sum(fi * feats[j], axis=0, keepdims=True)
                cs_ref[pl.ds(p, 1), :] = s
                p += 1
        cs_ref[pl.ds(NP, 32), :] = di
        cs_ref[pl.ds(NP + 32, 1), :] = jnp.zeros((1, BBLK), jnp.float32)
        t = jnp.maximum(
            lax.dot_general(wt1_ref[:, :], cs_ref[:, :],
                            (((1,), (0,)), ((), ())),
                            preferred_element_type=jnp.float32)
            + bt1_ref[:, :], 0.0)
        o = lax.dot_general(wt2_ref[:, :], t, (((1,), (0,)), ((), ())),
                            preferred_element_type=jnp.float32) \
            + bt2_ref[:, :]
        out_ref[:, :] = o

    grid = (B // BBLK,)
    full = lambda shp: pl.BlockSpec(shp, lambda i: (0, 0))
    return pl.pallas_call(
        tc_dense,
        grid=grid,
        in_specs=[
            pl.BlockSpec((F, BBLK, D), lambda i: (0, i, 0)),
            pl.BlockSpec((1, BBLK), lambda i: (0, i)),
            full((D, 1)), full((D, 1)),
            full((64, D)), full((64, 1)),
            full((32, 64)), full((32, 1)),
            full((32, CS_ROWS)), full((32, 1)),
            full((1, 32)), full((1, 1)),
        ],
        out_specs=pl.BlockSpec((1, BBLK), lambda i: (0, i)),
        out_shape=jax.ShapeDtypeStruct((1, B), jnp.float32),
        scratch_shapes=[pltpu.VMEM((CS_ROWS, BBLK), jnp.float32)],
    )


def kernel(x_cat, price, tables, W_price, b_price, W1, b1, W2, b2,
           Wt1, bt1, Wt2, bt2):
    F, B = x_cat.shape
    _, V, D = tables.shape
    NW = 32
    BBLK = 512

    tflat = tables.reshape(F * V, D)
    gidx = (x_cat.astype(jnp.int32)
            + (jnp.arange(F, dtype=jnp.int32) * V)[:, None]
            ).reshape(F, B // 128, 128)
    sc_gather = _build_sc_gather(F, B, V, D, NW)
    emb_t = sc_gather(gidx, tflat)

    NP = (F + 1) * F // 2
    Wt1p = jnp.pad(Wt1, ((0, 0), (0, 1)))
    tc = _build_tc_dense(F, B, D, BBLK)
    out2d = tc(emb_t, price.reshape(1, B), W_price, b_price.reshape(D, 1),
               W1, b1.reshape(64, 1), W2, b2.reshape(32, 1),
               Wt1p, bt1.reshape(32, 1), Wt2, bt2.reshape(1, 1))
    return out2d.reshape(B)
